# baseline (speedup 1.0000x reference)
import jax
import jax.numpy as jnp
from jax import lax
from jax.experimental import pallas as pl
from jax.experimental.pallas import tpu as pltpu
from jax.experimental.pallas import tpu_sc as plsc

_RATE = 2
_B, _T, _D = 16, 4096, 256
_TO = _T // _RATE
_C = 128
_NWORK = 32
_RPW = (_B * _TO) // _NWORK
_NCH = _RPW // _C
_L = 16


def _sc_body(view_hbm, lengths_hbm, out_hbm, nl_hbm,
             idx_v, gbuf, zbuf, lens_v, nl_v,
             gsem, fsem, psem, zsem):
    wid = lax.axis_index("s") * 2 + lax.axis_index("c")
    b = wid // 2
    h = wid % 2
    base = h * _RPW
    grow0 = b * _TO + base

    pltpu.sync_copy(lengths_hbm, lens_v)
    lane = lax.iota(jnp.int32, _L)
    myl = jnp.sum(jnp.where(lane == b, lens_v[...], 0))
    nl = myl // _RATE
    v = jnp.clip(nl - base, 0, _RPW)

    @pl.when(wid == 0)
    def _():
        nl_v[...] = lens_v[...] // _RATE
        pltpu.sync_copy(nl_v, nl_hbm)

    zeros16 = jnp.zeros((_L,), jnp.float32)

    @pl.loop(0, _C)
    def _(r):
        for j in range(_D // _L):
            zbuf[r, pl.ds(j * _L, _L)] = zeros16

    vk = [jnp.clip(v - k * _C, 0, _C) for k in range(_NCH)]
    orow = [grow0 + k * _C for k in range(_NCH)]

    def src_slice(k):
        return view_hbm.at[pl.ds(grow0 + k * _C, _C), pl.ds(_D, _D)]

    def dst_slice(k):
        return out_hbm.at[pl.ds(orow[k], _C)]

    for k in range(_NCH):
        full = vk[k] >= _C
        part = (vk[k] > 0) & (vk[k] < _C)

        @pl.when(full)
        def _(k=k):
            pltpu.async_copy(src_slice(k), dst_slice(k), fsem)

        @pl.when(part)
        def _(k=k):
            row0 = grow0 + k * _C
            for j in range(_C // _L):
                idx_v[pl.ds(j * _L, _L)] = row0 + j * _L + lane
            pltpu.async_copy(view_hbm.at[idx_v], gbuf, gsem).wait()

            @pl.loop(vk[k], _C)
            def _(r):
                for j in range(_D // _L):
                    gbuf[r, pl.ds(_D + j * _L, _L)] = zeros16

            pltpu.async_copy(
                gbuf.at[pl.ds(0, _C), pl.ds(_D, _D)], dst_slice(k), psem)

        @pl.when(vk[k] == 0)
        def _(k=k):
            pltpu.async_copy(zbuf, dst_slice(k), zsem)

    for k in range(_NCH):
        full = vk[k] >= _C
        part = (vk[k] > 0) & (vk[k] < _C)

        @pl.when(full)
        def _(k=k):
            pltpu.make_async_copy(src_slice(k), dst_slice(k), fsem).wait()

        @pl.when(part)
        def _(k=k):
            pltpu.make_async_copy(
                gbuf.at[pl.ds(0, _C), pl.ds(_D, _D)], dst_slice(k), psem).wait()

        @pl.when(vk[k] == 0)
        def _(k=k):
            pltpu.make_async_copy(zbuf, dst_slice(k), zsem).wait()


def kernel(padded, lengths):
    view = padded.reshape(_B * _TO, _RATE * _D)
    mesh = plsc.VectorSubcoreMesh(core_axis_name="c", subcore_axis_name="s")
    out2d, nl = pl.kernel(
        _sc_body,
        out_type=(
            jax.ShapeDtypeStruct((_B * _TO, _D), jnp.float32),
            jax.ShapeDtypeStruct((_B,), jnp.int32),
        ),
        mesh=mesh,
        compiler_params=pltpu.CompilerParams(needs_layout_passes=False),
        scratch_types=(
            pltpu.VMEM((_C,), jnp.int32),
            pltpu.VMEM((_C, _RATE * _D), jnp.float32),
            pltpu.VMEM((_C, _D), jnp.float32),
            pltpu.VMEM((_L,), jnp.int32),
            pltpu.VMEM((_L,), jnp.int32),
            pltpu.SemaphoreType.DMA,
            pltpu.SemaphoreType.DMA,
            pltpu.SemaphoreType.DMA,
            pltpu.SemaphoreType.DMA,
        ),
    )(view, lengths)
    return out2d.reshape(_B, _TO, _D), nl


# speedup vs baseline: 10.6313x; 10.6313x over previous
import jax
import jax.numpy as jnp
from jax import lax
from jax.experimental import pallas as pl
from jax.experimental.pallas import tpu as pltpu
from jax.experimental.pallas import tpu_sc as plsc

_RATE = 2
_B, _T, _D = 16, 4096, 256
_TO = _T // _RATE
_C = 128
_NWORK = 32
_RPW = (_B * _TO) // _NWORK
_NCH = _RPW // _C
_L = 16
_NBUF = 3
_ZR = 64


def _sc_body(padded_hbm, lengths_hbm, out_hbm, nl_hbm,
             idx_v, gbuf0, gbuf1, gbuf2, zbuf, lens_v, nl_v,
             gsem0, gsem1, gsem2, osem0, osem1, osem2, zsem):
    gbuf = (gbuf0, gbuf1, gbuf2)
    gsem = (gsem0, gsem1, gsem2)
    osem = (osem0, osem1, osem2)

    wid = lax.axis_index("s") * 2 + lax.axis_index("c")
    b = wid // 2
    h = wid % 2
    base = h * _RPW
    grow0 = b * _TO + base

    pltpu.sync_copy(lengths_hbm, lens_v)
    lane = lax.iota(jnp.int32, _L)
    myl = jnp.sum(jnp.where(lane == b, lens_v[...], 0))
    nl = myl // _RATE
    v = jnp.clip(nl - base, 0, _RPW)

    @pl.when(wid == 0)
    def _():
        nl_v[...] = lens_v[...] // _RATE
        pltpu.sync_copy(nl_v, nl_hbm)

    row0 = b * _T + 2 * base + 1
    for j in range(_RPW // _L):
        idx_v[pl.ds(j * _L, _L)] = row0 + 2 * (j * _L + lane)

    zeros16 = jnp.zeros((_L,), jnp.float32)

    @pl.loop(0, _ZR)
    def _(r):
        for j in range(_D // _L):
            zbuf[r, pl.ds(j * _L, _L)] = zeros16

    vk = [jnp.clip(v - k * _C, 0, _C) for k in range(_NCH)]
    orow = [grow0 + k * _C for k in range(_NCH)]

    for k in range(_NCH):
        @pl.when(vk[k] == 0)
        def _(k=k):
            pltpu.async_copy(zbuf, out_hbm.at[pl.ds(orow[k], _ZR)], zsem)
            pltpu.async_copy(zbuf, out_hbm.at[pl.ds(orow[k] + _ZR, _ZR)], zsem)

    def gather_desc(k):
        s = k % _NBUF
        return pltpu.make_async_copy(
            padded_hbm.at[idx_v.at[pl.ds(k * _C, _C)]], gbuf[s], gsem[s])

    def out_desc(k):
        s = k % _NBUF
        return pltpu.make_async_copy(
            gbuf[s], out_hbm.at[pl.ds(orow[k], _C)], osem[s])

    def finish_gather_start_out(k):
        gather_desc(k).wait()
        s = k % _NBUF

        @pl.when(vk[k] < _C)
        def _():
            @pl.loop(vk[k], _C)
            def _(r):
                for j in range(_D // _L):
                    gbuf[s][r, pl.ds(j * _L, _L)] = zeros16

        out_desc(k).start()

    for k in range(_NCH):
        @pl.when(vk[k] > 0)
        def _(k=k):
            if k >= _NBUF:
                @pl.when(vk[k - _NBUF] > 0)
                def _():
                    out_desc(k - _NBUF).wait()
            gather_desc(k).start()
        if k >= 1:
            @pl.when(vk[k - 1] > 0)
            def _(k=k):
                finish_gather_start_out(k - 1)

    @pl.when(vk[_NCH - 1] > 0)
    def _():
        finish_gather_start_out(_NCH - 1)

    for k in range(max(_NCH - _NBUF, 0), _NCH):
        @pl.when(vk[k] > 0)
        def _(k=k):
            out_desc(k).wait()

    for k in range(_NCH):
        @pl.when(vk[k] == 0)
        def _(k=k):
            pltpu.make_async_copy(
                zbuf, out_hbm.at[pl.ds(orow[k], _ZR)], zsem).wait()
            pltpu.make_async_copy(
                zbuf, out_hbm.at[pl.ds(orow[k] + _ZR, _ZR)], zsem).wait()


def kernel(padded, lengths):
    padded2d = padded.reshape(_B * _T, _D)
    mesh = plsc.VectorSubcoreMesh(core_axis_name="c", subcore_axis_name="s")
    out2d, nl = pl.kernel(
        _sc_body,
        out_type=(
            jax.ShapeDtypeStruct((_B * _TO, _D), jnp.float32),
            jax.ShapeDtypeStruct((_B,), jnp.int32),
        ),
        mesh=mesh,
        compiler_params=pltpu.CompilerParams(needs_layout_passes=False),
        scratch_types=(
            pltpu.VMEM((_RPW,), jnp.int32),
            pltpu.VMEM((_C, _D), jnp.float32),
            pltpu.VMEM((_C, _D), jnp.float32),
            pltpu.VMEM((_C, _D), jnp.float32),
            pltpu.VMEM((_ZR, _D), jnp.float32),
            pltpu.VMEM((_L,), jnp.int32),
            pltpu.VMEM((_L,), jnp.int32),
            pltpu.SemaphoreType.DMA,
            pltpu.SemaphoreType.DMA,
            pltpu.SemaphoreType.DMA,
            pltpu.SemaphoreType.DMA,
            pltpu.SemaphoreType.DMA,
            pltpu.SemaphoreType.DMA,
            pltpu.SemaphoreType.DMA,
        ),
    )(padded2d, lengths)
    return out2d.reshape(_B, _TO, _D), nl


# speedup vs baseline: 11.0363x; 1.0381x over previous
import jax
import jax.numpy as jnp
from jax import lax
from jax.experimental import pallas as pl
from jax.experimental.pallas import tpu as pltpu
from jax.experimental.pallas import tpu_sc as plsc

_RATE = 2
_B, _T, _D = 16, 4096, 256
_TO = _T // _RATE
_C = 128
_NWORK = 32
_NCHG = (_B * _TO) // _C
_SLOTS = _NCHG // _NWORK
_JPB = _TO // _C
_L = 16
_NBUF = 3
_ZR = 64
_PERM = 173


def _sc_body(padded_hbm, lengths_hbm, out_hbm, nl_hbm,
             idx0, idx1, idx2, gbuf0, gbuf1, gbuf2, zbuf, lens_v, nl_v,
             gsem0, gsem1, gsem2, osem0, osem1, osem2, zsem):
    idxb = (idx0, idx1, idx2)
    gbuf = (gbuf0, gbuf1, gbuf2)
    gsem = (gsem0, gsem1, gsem2)
    osem = (osem0, osem1, osem2)

    wid = lax.axis_index("s") * 2 + lax.axis_index("c")

    pltpu.sync_copy(lengths_hbm, lens_v)
    lane = lax.iota(jnp.int32, _L)
    lens = lens_v[...]

    @pl.when(wid == 0)
    def _():
        nl_v[...] = lens // _RATE
        pltpu.sync_copy(nl_v, nl_hbm)

    zeros16 = jnp.zeros((_L,), jnp.float32)

    @pl.loop(0, _ZR)
    def _(r):
        for j in range(_D // _L):
            zbuf[r, pl.ds(j * _L, _L)] = zeros16

    kg, bb, vk, orow, row0 = [], [], [], [], []
    for i in range(_SLOTS):
        k = (_PERM * (_SLOTS * wid + i)) % _NCHG
        b = k // _JPB
        j = k % _JPB
        nl = jnp.sum(jnp.where(lane == b, lens, 0)) // _RATE
        kg.append(k)
        bb.append(b)
        vk.append(jnp.clip(nl - j * _C, 0, _C))
        orow.append(k * _C)
        row0.append(b * _T + 2 * (j * _C) + 1)

    for i in range(_SLOTS):
        @pl.when(vk[i] == 0)
        def _(i=i):
            pltpu.async_copy(zbuf, out_hbm.at[pl.ds(orow[i], _ZR)], zsem)
            pltpu.async_copy(zbuf, out_hbm.at[pl.ds(orow[i] + _ZR, _ZR)], zsem)

    def gather_desc(i):
        s = i % _NBUF
        return pltpu.make_async_copy(
            padded_hbm.at[idxb[s]], gbuf[s], gsem[s])

    def out_desc(i):
        s = i % _NBUF
        return pltpu.make_async_copy(
            gbuf[s], out_hbm.at[pl.ds(orow[i], _C)], osem[s])

    def start_gather(i):
        s = i % _NBUF
        for j in range(_C // _L):
            idxb[s][pl.ds(j * _L, _L)] = row0[i] + 2 * (j * _L + lane)
        gather_desc(i).start()

    def finish_gather_start_out(i):
        gather_desc(i).wait()
        s = i % _NBUF

        @pl.when(vk[i] < _C)
        def _():
            @pl.loop(vk[i], _C)
            def _(r):
                for j in range(_D // _L):
                    gbuf[s][r, pl.ds(j * _L, _L)] = zeros16

        out_desc(i).start()

    for i in range(_SLOTS):
        if i >= _NBUF:
            @pl.when(vk[i - _NBUF] > 0)
            def _(i=i):
                out_desc(i - _NBUF).wait()

        @pl.when(vk[i] > 0)
        def _(i=i):
            start_gather(i)
        if i >= 1:
            @pl.when(vk[i - 1] > 0)
            def _(i=i):
                finish_gather_start_out(i - 1)

    @pl.when(vk[_SLOTS - 1] > 0)
    def _():
        finish_gather_start_out(_SLOTS - 1)

    for i in range(max(_SLOTS - _NBUF, 0), _SLOTS):
        @pl.when(vk[i] > 0)
        def _(i=i):
            out_desc(i).wait()

    for i in range(_SLOTS):
        @pl.when(vk[i] == 0)
        def _(i=i):
            pltpu.make_async_copy(
                zbuf, out_hbm.at[pl.ds(orow[i], _ZR)], zsem).wait()
            pltpu.make_async_copy(
                zbuf, out_hbm.at[pl.ds(orow[i] + _ZR, _ZR)], zsem).wait()


def kernel(padded, lengths):
    padded2d = padded.reshape(_B * _T, _D)
    mesh = plsc.VectorSubcoreMesh(core_axis_name="c", subcore_axis_name="s")
    out2d, nl = pl.kernel(
        _sc_body,
        out_type=(
            jax.ShapeDtypeStruct((_B * _TO, _D), jnp.float32),
            jax.ShapeDtypeStruct((_B,), jnp.int32),
        ),
        mesh=mesh,
        compiler_params=pltpu.CompilerParams(needs_layout_passes=False),
        scratch_types=(
            pltpu.VMEM((_C,), jnp.int32),
            pltpu.VMEM((_C,), jnp.int32),
            pltpu.VMEM((_C,), jnp.int32),
            pltpu.VMEM((_C, _D), jnp.float32),
            pltpu.VMEM((_C, _D), jnp.float32),
            pltpu.VMEM((_C, _D), jnp.float32),
            pltpu.VMEM((_ZR, _D), jnp.float32),
            pltpu.VMEM((_L,), jnp.int32),
            pltpu.VMEM((_L,), jnp.int32),
            pltpu.SemaphoreType.DMA,
            pltpu.SemaphoreType.DMA,
            pltpu.SemaphoreType.DMA,
            pltpu.SemaphoreType.DMA,
            pltpu.SemaphoreType.DMA,
            pltpu.SemaphoreType.DMA,
            pltpu.SemaphoreType.DMA,
        ),
    )(padded2d, lengths)
    return out2d.reshape(_B, _TO, _D), nl
